# baseline (speedup 1.0000x reference)
import functools

import jax
import jax.numpy as jnp
from jax import lax
from jax.experimental import pallas as pl
from jax.experimental.pallas import tpu as pltpu
from jax.experimental.pallas import tpu_sc as plsc

B = 2
C_IN = 96
H = 112
W = 112
R = 2
KK = 9
C1D = C_IN * R * R
C_OUT = 96
T = (H // R) * (W // R)
ROWS = 448
NTILE = T // ROWS

NC, NS = 2, 16
NW = NC * NS
IDX_TOTAL = B * T * KK
CHUNK = 128
NCHUNK = 8
PER_W = NCHUNK * CHUNK
IDXB_TOTAL = T * KK
IDXB_PAD = NW * PER_W
GROUP = 4
GROWS = GROUP * CHUNK

SIM_DTYPE = jnp.bfloat16


def _prep_body(flatT_ref, flat_ref, wv1_ref, knr_ref, knT_ref, hT_ref):
    ft = flatT_ref[0]
    n_r = jnp.sqrt(jnp.sum(ft * ft, axis=1, keepdims=True))
    knr_ref[0] = (ft / (n_r + 1e-8)).astype(SIM_DTYPE)
    f = flat_ref[0]
    n_c = jnp.sqrt(jnp.sum(f * f, axis=0, keepdims=True))
    knT_ref[0] = (f / (n_c + 1e-8)).astype(SIM_DTYPE)
    h = lax.dot_general(ft, wv1_ref[...], (((1,), (1,)), ((), ())),
                        preferred_element_type=jnp.float32)
    hT_ref[0] = jnp.maximum(h, 0.0)


def _prep(flatT, flat, w_v1):
    return pl.pallas_call(
        _prep_body,
        grid=(B,),
        in_specs=[
            pl.BlockSpec((1, T, C1D), lambda b: (b, 0, 0)),
            pl.BlockSpec((1, C1D, T), lambda b: (b, 0, 0)),
            pl.BlockSpec((8, C1D), lambda b: (0, 0)),
        ],
        out_specs=[
            pl.BlockSpec((1, T, C1D), lambda b: (b, 0, 0)),
            pl.BlockSpec((1, C1D, T), lambda b: (b, 0, 0)),
            pl.BlockSpec((1, T, 8), lambda b: (b, 0, 0)),
        ],
        out_shape=[
            jax.ShapeDtypeStruct((B, T, C1D), SIM_DTYPE),
            jax.ShapeDtypeStruct((B, C1D, T), SIM_DTYPE),
            jax.ShapeDtypeStruct((B, T, 8), jnp.float32),
        ],
    )(flatT, flat, w_v1)


def _make_topk_body(base):
    def _topk_body(knr_ref, knT_ref, attn_ref, topi_ref):
        i = pl.program_id(0)
        rows = knr_ref[...]
        cols = knT_ref[...]
        sim = jnp.dot(rows, cols, preferred_element_type=jnp.float32)
        col_ids = lax.broadcasted_iota(jnp.int32, (ROWS, T), 1)
        row_ids = lax.broadcasted_iota(jnp.int32, (ROWS, T), 0) + i * ROWS
        sim = jnp.where(col_ids == row_ids, 1.1, sim)
        vals, idxs = [], []
        for _ in range(KK):
            m = jnp.max(sim, axis=1, keepdims=True)
            hit = sim == m
            idx = jnp.min(jnp.where(hit, col_ids, T), axis=1,
                          keepdims=True)
            vals.append(m)
            idxs.append(idx)
            sim = jnp.where(col_ids == idx, -3.0, sim)
        topv = jnp.concatenate(vals, axis=1)
        topi = jnp.concatenate(idxs, axis=1)
        e = jnp.exp(topv - jnp.max(topv, axis=1, keepdims=True))
        attn_ref[...] = e / jnp.sum(e, axis=1, keepdims=True)
        topi_ref[...] = topi + base
    return _topk_body


def _topk_batch(knr_b, knT_b, base):
    return pl.pallas_call(
        _make_topk_body(base),
        grid=(NTILE,),
        in_specs=[
            pl.BlockSpec((ROWS, C1D), lambda i: (i, 0)),
            pl.BlockSpec((C1D, T), lambda i: (0, 0)),
        ],
        out_specs=[
            pl.BlockSpec((ROWS, KK), lambda i: (i, 0)),
            pl.BlockSpec((ROWS, KK), lambda i: (i, 0)),
        ],
        out_shape=[
            jax.ShapeDtypeStruct((T, KK), jnp.float32),
            jax.ShapeDtypeStruct((T, KK), jnp.int32),
        ],
    )(knr_b, knT_b)


def _sc_gather_body(table_hbm, idx_hbm, out_hbm, table_v, idx_v, out_v):
    wid = lax.axis_index("s") * NC + lax.axis_index("c")
    pltpu.sync_copy(idx_hbm.at[pl.ds(wid * PER_W, PER_W)], idx_v)
    pltpu.sync_copy(table_hbm, table_v)
    lane = lax.iota(jnp.int32, 16)
    out_base = wid * PER_W * 8

    def chunk_body(c, carry):
        for jv in range(CHUNK // 16):
            j16 = lane + jv * 16
            idx16 = plsc.load_gather(idx_v, [c * CHUNK + j16])
            a16 = idx16 * 8
            o16 = j16 * 8
            for ch in range(8):
                vals = plsc.load_gather(table_v, [a16 + ch])
                plsc.store_scatter(out_v, [o16 + ch], vals)
        pltpu.sync_copy(
            out_v,
            out_hbm.at[pl.ds(out_base + c * (CHUNK * 8), CHUNK * 8)])
        return carry

    lax.fori_loop(0, NCHUNK, chunk_body, 0)


@functools.cache
def _make_sc_gather():
    return pl.kernel(
        _sc_gather_body,
        out_type=jax.ShapeDtypeStruct((IDXB_PAD * 8,), jnp.float32),
        mesh=plsc.VectorSubcoreMesh(core_axis_name="c",
                                    subcore_axis_name="s"),
        compiler_params=pltpu.CompilerParams(needs_layout_passes=False),
        scratch_types=[
            pltpu.VMEM((B * T * 8,), jnp.float32),
            pltpu.VMEM((PER_W,), jnp.int32),
            pltpu.VMEM((CHUNK * 8,), jnp.float32),
        ],
    )


def _sc_gather(table, idx_pad):
    return _make_sc_gather()(table, idx_pad)


def _finish_body(gath_ref, attn_ref, wc9_ref, wv2_ref, wo_ref, bc_ref,
                 out_ref):
    wo = wo_ref[...]
    blocks = []
    for k in range(KK):
        wck = wc9_ref[k * C_OUT:(k + 1) * C_OUT, :]
        mkT = lax.dot_general(wv2_ref[...], wck, (((0,), (1,)), ((), ())),
                              preferred_element_type=jnp.float32)
        blk = lax.dot_general(mkT, wo, (((1,), (1,)), ((), ())),
                              preferred_element_type=jnp.float32)
        blocks.append(blk)
    wbig = jnp.concatenate(blocks, axis=0)
    jj = lax.broadcasted_iota(jnp.int32, (KK, KK * 8), 1)
    kk = lax.broadcasted_iota(jnp.int32, (KK, KK * 8), 0)
    expand = jnp.where(jj // 8 == kk, 1.0, 0.0)
    attn16 = jnp.dot(attn_ref[...], expand,
                     preferred_element_type=jnp.float32)
    g16 = gath_ref[...] * attn16
    out = jnp.dot(g16, wbig, preferred_element_type=jnp.float32)
    bias2 = lax.dot_general(bc_ref[...], wo, (((1,), (1,)), ((), ())),
                            preferred_element_type=jnp.float32)
    out_ref[...] = out + bias2


def _finish(gath144, attn2, wc9, w_v2, w_o, b_conv2):
    return pl.pallas_call(
        _finish_body,
        out_shape=jax.ShapeDtypeStruct((B * T, C_OUT), jnp.float32),
    )(gath144, attn2, wc9, w_v2, w_o, b_conv2)


def kernel(x, W_v1, W_v2, W_conv, b_conv, W_o):
    xs = x.reshape(B, C_IN, H // R, R, W // R, R)
    xs = xs.transpose(0, 1, 3, 5, 2, 4)
    flat = xs.reshape(B, C1D, T)
    flatT = flat.transpose(0, 2, 1)

    knr, knT, hT16 = _prep(flatT, flat, W_v1)
    table = hT16.reshape(B * T * 8)

    gs, ats = [], []
    for b in range(B):
        attn_b, topig_b = _topk_batch(knr[b], knT[b], b * T)
        idx_b = jnp.concatenate(
            [topig_b.reshape(IDXB_TOTAL),
             jnp.zeros((IDXB_PAD - IDXB_TOTAL,), jnp.int32)])
        gath_b = _sc_gather(table, idx_b)
        gs.append(gath_b.reshape(IDXB_PAD, 8)[:IDXB_TOTAL])
        ats.append(attn_b)
    gath144 = jnp.concatenate(gs, axis=0).reshape(B * T, KK * 8)
    attn = jnp.concatenate(ats, axis=0)

    wc9 = jnp.transpose(W_conv, (2, 0, 1)).reshape(KK * C_OUT, C1D)
    out = _finish(gath144, attn, wc9, W_v2, W_o,
                  b_conv.reshape(1, C_OUT))
    return out.reshape(B, T, C_OUT).transpose(0, 2, 1).reshape(
        B, C_OUT, H // R, W // R)


# speedup vs baseline: 1.2498x; 1.2498x over previous
import functools

import jax
import jax.numpy as jnp
from jax import lax
from jax.experimental import pallas as pl
from jax.experimental.pallas import tpu as pltpu
from jax.experimental.pallas import tpu_sc as plsc

B = 2
C_IN = 96
H = 112
W = 112
R = 2
KK = 9
C1D = C_IN * R * R
C_OUT = 96
T = (H // R) * (W // R)
ROWS = 448
NTILE = T // ROWS

NC, NS = 2, 16
NW = NC * NS
IDX_TOTAL = B * T * KK
CHUNK = 128
NCHUNK = 16
PER_W = NCHUNK * CHUNK
IDX_PAD = NW * PER_W
GROUP = 4
GROWS = GROUP * CHUNK

SIM_DTYPE = jnp.bfloat16


def _prep_body(flat_ref, wv1_ref, knT_ref, hT_ref):
    f = flat_ref[0]
    n_c = jnp.sqrt(jnp.sum(f * f, axis=0, keepdims=True))
    knT_ref[0] = (f / (n_c + 1e-8)).astype(SIM_DTYPE)
    h = lax.dot_general(f, wv1_ref[...], (((0,), (1,)), ((), ())),
                        preferred_element_type=jnp.float32)
    hT_ref[0] = jnp.maximum(h, 0.0)


def _prep(flat, w_v1):
    return pl.pallas_call(
        _prep_body,
        grid=(B,),
        in_specs=[
            pl.BlockSpec((1, C1D, T), lambda b: (b, 0, 0)),
            pl.BlockSpec((8, C1D), lambda b: (0, 0)),
        ],
        out_specs=[
            pl.BlockSpec((1, C1D, T), lambda b: (b, 0, 0)),
            pl.BlockSpec((1, T, 8), lambda b: (b, 0, 0)),
        ],
        out_shape=[
            jax.ShapeDtypeStruct((B, C1D, T), SIM_DTYPE),
            jax.ShapeDtypeStruct((B, T, 8), jnp.float32),
        ],
    )(flat, w_v1)


def _topk_body(knr_ref, knT_ref, attn_ref, topi_ref):
    b = pl.program_id(0)
    i = pl.program_id(1)
    rows = knr_ref[0]
    cols = knT_ref[0]
    sim = jnp.dot(rows, cols, preferred_element_type=jnp.float32)
    col_ids = lax.broadcasted_iota(jnp.int32, (ROWS, T), 1)
    row_ids = lax.broadcasted_iota(jnp.int32, (ROWS, T), 0) + i * ROWS
    sim = jnp.where(col_ids == row_ids, 1.1, sim)
    vals, idxs = [], []
    for _ in range(KK):
        m = jnp.max(sim, axis=1, keepdims=True)
        hit = sim == m
        idx = jnp.min(jnp.where(hit, col_ids, T), axis=1, keepdims=True)
        vals.append(m)
        idxs.append(idx)
        sim = jnp.where(col_ids == idx, -3.0, sim)
    topv = jnp.concatenate(vals, axis=1)
    topi = jnp.concatenate(idxs, axis=1)
    e = jnp.exp(topv - jnp.max(topv, axis=1, keepdims=True))
    attn_ref[0] = e / jnp.sum(e, axis=1, keepdims=True)
    topi_ref[0] = topi + b * T


def _topk(knr, knT):
    return pl.pallas_call(
        _topk_body,
        grid=(B, NTILE),
        in_specs=[
            pl.BlockSpec((1, ROWS, C1D), lambda b, i: (b, i, 0)),
            pl.BlockSpec((1, C1D, T), lambda b, i: (b, 0, 0)),
        ],
        out_specs=[
            pl.BlockSpec((1, ROWS, KK), lambda b, i: (b, i, 0)),
            pl.BlockSpec((1, ROWS, KK), lambda b, i: (b, i, 0)),
        ],
        out_shape=[
            jax.ShapeDtypeStruct((B, T, KK), jnp.float32),
            jax.ShapeDtypeStruct((B, T, KK), jnp.int32),
        ],
    )(knr, knT)


def _sc_gather_body(table_hbm, idx_hbm, out_hbm, table_v, idx_v, out_v):
    wid = lax.axis_index("s") * NC + lax.axis_index("c")
    pltpu.sync_copy(idx_hbm.at[pl.ds(wid * PER_W, PER_W)], idx_v)
    pltpu.sync_copy(table_hbm, table_v)
    lane = lax.iota(jnp.int32, 16)
    out_base = wid * PER_W * 8

    def chunk_body(c, carry):
        for jv in range(CHUNK // 16):
            j16 = lane + jv * 16
            idx16 = plsc.load_gather(idx_v, [c * CHUNK + j16])
            a16 = idx16 * 8
            o16 = j16 * 8
            for ch in range(8):
                vals = plsc.load_gather(table_v, [a16 + ch])
                plsc.store_scatter(out_v, [o16 + ch], vals)
        pltpu.sync_copy(
            out_v,
            out_hbm.at[pl.ds(out_base + c * (CHUNK * 8), CHUNK * 8)])
        return carry

    lax.fori_loop(0, NCHUNK, chunk_body, 0)


@functools.cache
def _make_sc_gather():
    return pl.kernel(
        _sc_gather_body,
        out_type=jax.ShapeDtypeStruct((IDX_PAD * 8,), jnp.float32),
        mesh=plsc.VectorSubcoreMesh(core_axis_name="c",
                                    subcore_axis_name="s"),
        compiler_params=pltpu.CompilerParams(needs_layout_passes=False),
        scratch_types=[
            pltpu.VMEM((B * T * 8,), jnp.float32),
            pltpu.VMEM((PER_W,), jnp.int32),
            pltpu.VMEM((CHUNK * 8,), jnp.float32),
        ],
    )


def _sc_gather(table, idx_pad):
    return _make_sc_gather()(table, idx_pad)


def _finish_body(gath_ref, attn_ref, wc9_ref, wv2_ref, wo_ref, bc_ref,
                 out_ref):
    wo = wo_ref[...]
    blocks = []
    for k in range(KK):
        wck = wc9_ref[k * C_OUT:(k + 1) * C_OUT, :]
        mkT = lax.dot_general(wv2_ref[...], wck, (((0,), (1,)), ((), ())),
                              preferred_element_type=jnp.float32)
        blk = lax.dot_general(mkT, wo, (((1,), (1,)), ((), ())),
                              preferred_element_type=jnp.float32)
        blocks.append(blk)
    wbig = jnp.concatenate(blocks, axis=0)
    jj = lax.broadcasted_iota(jnp.int32, (KK, KK * 8), 1)
    kk = lax.broadcasted_iota(jnp.int32, (KK, KK * 8), 0)
    expand = jnp.where(jj // 8 == kk, 1.0, 0.0)
    attn16 = jnp.dot(attn_ref[...], expand,
                     preferred_element_type=jnp.float32)
    g16 = gath_ref[...] * attn16
    out = jnp.dot(g16, wbig, preferred_element_type=jnp.float32)
    bias2 = lax.dot_general(bc_ref[...], wo, (((1,), (1,)), ((), ())),
                            preferred_element_type=jnp.float32)
    out_ref[...] = out + bias2


def _finish(gath144, attn2, wc9, w_v2, w_o, b_conv2):
    return pl.pallas_call(
        _finish_body,
        out_shape=jax.ShapeDtypeStruct((B * T, C_OUT), jnp.float32),
    )(gath144, attn2, wc9, w_v2, w_o, b_conv2)


def kernel(x, W_v1, W_v2, W_conv, b_conv, W_o):
    xs = x.reshape(B, C_IN, H // R, R, W // R, R)
    xs = xs.transpose(0, 1, 3, 5, 2, 4)
    flat = xs.reshape(B, C1D, T)

    knT, hT16 = _prep(flat, W_v1)
    knr = knT.transpose(0, 2, 1)
    attn, topig = _topk(knr, knT)

    idx_flat = topig.reshape(IDX_TOTAL)
    idx_pad = jnp.concatenate(
        [idx_flat, jnp.zeros((IDX_PAD - IDX_TOTAL,), jnp.int32)])
    table = hT16.reshape(B * T * 8)

    gath = _sc_gather(table, idx_pad)
    gath144 = gath.reshape(IDX_PAD, 8)[:IDX_TOTAL].reshape(
        B * T, KK * 8)

    wc9 = jnp.transpose(W_conv, (2, 0, 1)).reshape(KK * C_OUT, C1D)
    out = _finish(gath144, attn.reshape(B * T, KK), wc9, W_v2, W_o,
                  b_conv.reshape(1, C_OUT))
    return out.reshape(B, T, C_OUT).transpose(0, 2, 1).reshape(
        B, C_OUT, H // R, W // R)
